# Initial kernel scaffold; baseline (speedup 1.0000x reference)
import jax
import jax.numpy as jnp
from jax.experimental import pallas as pl


def kernel(input, edge_index, weight, bias):
    raise NotImplementedError("write your pallas kernel here")


# speedup vs baseline: 3.2936x; 3.2936x over previous
import functools

import jax
import jax.numpy as jnp
from jax import lax
from jax.experimental import pallas as pl
from jax.experimental.pallas import tpu as pltpu
from jax.experimental.pallas import tpu_sc as plsc

F = 128
CHUNK = 128
MM_BLK = 1000


def _matmul_body(x_ref, w_ref, out_ref):
    out_ref[...] = jnp.dot(x_ref[...], w_ref[...],
                           preferred_element_type=jnp.float32)


def _matmul(x, w):
    n, f = x.shape
    return pl.pallas_call(
        _matmul_body,
        grid=(n // MM_BLK,),
        in_specs=[
            pl.BlockSpec((MM_BLK, f), lambda i: (i, 0)),
            pl.BlockSpec((f, f), lambda i: (0, 0)),
        ],
        out_specs=pl.BlockSpec((MM_BLK, f), lambda i: (i, 0)),
        out_shape=jax.ShapeDtypeStruct((n, f), jnp.float32),
    )(x, w)


def _combine_body(p0_ref, p1_ref, b_ref, out_ref):
    out_ref[...] = p0_ref[0] + p1_ref[0] + b_ref[...]


def _combine(partials, bias, n):
    f = partials.shape[2]
    return pl.pallas_call(
        _combine_body,
        grid=(n // MM_BLK,),
        in_specs=[
            pl.BlockSpec((1, MM_BLK, f), lambda i: (0, i, 0)),
            pl.BlockSpec((1, MM_BLK, f), lambda i: (1, i, 0)),
            pl.BlockSpec((1, f), lambda i: (0, 0)),
        ],
        out_specs=pl.BlockSpec((MM_BLK, f), lambda i: (i, 0)),
        out_shape=jax.ShapeDtypeStruct((n, f), jnp.float32),
    )(partials, partials, bias.reshape(1, f))


@functools.cache
def _make_sc_agg(n_nodes, nchunks, f):
    info = plsc.get_sparse_core_info()
    nc, ns = info.num_cores, info.num_subcores
    nw = nc * ns
    cpw = nchunks // nw
    zrows = ((n_nodes // ns) + 8 + 7) // 8 * 8
    acc_rows = ns * zrows

    mesh = plsc.VectorSubcoreMesh(core_axis_name="c", subcore_axis_name="s")

    @functools.partial(
        pl.kernel,
        mesh=mesh,
        out_type=jax.ShapeDtypeStruct((nc, acc_rows, f), jnp.float32),
        scratch_types=[
            pltpu.VMEM((cpw, CHUNK), jnp.int32),
            pltpu.VMEM((cpw, CHUNK), jnp.int32),
            pltpu.VMEM((CHUNK, f), jnp.float32),
            pltpu.VMEM_SHARED((acc_rows, f), jnp.float32),
            pltpu.SemaphoreType.DMA,
        ],
    )
    def agg(sup_hbm, src_hbm, dst_hbm, zero_hbm, out_hbm,
            src_v, dst_v, rows_v, acc, sem):
        cid = lax.axis_index("c")
        sid = lax.axis_index("s")
        wid = sid * nc + cid
        pltpu.sync_copy(zero_hbm, acc.at[pl.ds(sid * zrows, zrows)])
        pltpu.sync_copy(src_hbm.at[pl.ds(wid * cpw, cpw)], src_v)
        pltpu.sync_copy(dst_hbm.at[pl.ds(wid * cpw, cpw)], dst_v)
        plsc.subcore_barrier()

        def body(j, carry):
            pltpu.async_copy(sup_hbm.at[src_v.at[j]], rows_v, sem).wait()
            pltpu.sync_copy(rows_v, acc.at[dst_v.at[j]], add=True)
            return carry

        lax.fori_loop(0, cpw, body, 0)
        plsc.subcore_barrier()
        pltpu.sync_copy(acc.at[pl.ds(sid * zrows, zrows)],
                        out_hbm.at[cid, pl.ds(sid * zrows, zrows)])

    return agg


def kernel(input, edge_index, weight, bias):
    n, f = input.shape
    e = edge_index.shape[1]
    support = _matmul(input, weight)

    ei = edge_index.astype(jnp.int32)
    nw = 32
    nchunks = -(-e // CHUNK)
    nchunks = -(-nchunks // (nw * 8)) * (nw * 8)
    epad = nchunks * CHUNK
    src = jnp.concatenate(
        [ei[1], jnp.zeros((epad - e,), jnp.int32)]).reshape(nchunks, CHUNK)
    dst = jnp.concatenate(
        [ei[0], jnp.full((epad - e,), n, jnp.int32)]).reshape(nchunks, CHUNK)

    agg = _make_sc_agg(n, nchunks, f)
    zrows = ((n // 16) + 8 + 7) // 8 * 8
    zeros = jnp.zeros((zrows, f), jnp.float32)
    partials = agg(support, src, dst, zeros)
    return _combine(partials, bias, n)
